# baseline (speedup 1.0000x reference)
import functools

import jax
import jax.numpy as jnp
from jax import lax
from jax.experimental import pallas as pl
from jax.experimental.pallas import tpu as pltpu
from jax.experimental.pallas import tpu_sc as plsc

B = 16384
D = 64
NC = 2
NS = 16
NW = NC * NS
BPW = B // NW
CH = 128
NCH = BPW // CH
L = 16
NSEM = 8


def _body(uidx_hbm, iidx_hbm, utab_hbm, itab_hbm, out_hbm,
          uidx_v, iidx_v, urows_v, irows_v, out_v, sems):
    wid = lax.axis_index("s") * NC + lax.axis_index("c")

    pltpu.sync_copy(uidx_hbm.at[wid], uidx_v)
    pltpu.sync_copy(iidx_hbm.at[wid], iidx_v)

    lane = lax.iota(jnp.int32, L)

    def lane_sum(x):
        for k in (8, 4, 2, 1):
            x = x + x.at[lane ^ k].get(mode="promise_in_bounds", unique_indices=True)
        return x

    def fire(c):
        base = c * CH

        @pl.loop(0, CH // L)
        def _fire(g):
            uvec = uidx_v[pl.ds(base + g * L, L)]
            ivec = iidx_v[pl.ds(base + g * L, L)]
            for l in range(L):
                dst = g * L + l
                pltpu.async_copy(utab_hbm.at[uvec[l]], urows_v.at[dst],
                                 sems.at[(2 * l) % NSEM])
                pltpu.async_copy(itab_hbm.at[ivec[l]], irows_v.at[dst],
                                 sems.at[(2 * l + 1) % NSEM])

    def drain_and_compute(c):
        base = c * CH

        @pl.loop(0, CH // L)
        def _drain(g):
            for l in range(L):
                pltpu.make_async_copy(utab_hbm.at[0], urows_v.at[0],
                                      sems.at[(2 * l) % NSEM]).wait()
                pltpu.make_async_copy(itab_hbm.at[0], irows_v.at[0],
                                      sems.at[(2 * l + 1) % NSEM]).wait()

        @pl.loop(0, CH // L)
        def _group(g):
            gbase = g * L
            res = jnp.zeros((L,), jnp.float32)
            for l in range(L):
                i = gbase + l
                acc = urows_v[i, pl.ds(0, L)] * irows_v[i, pl.ds(0, L)]
                for k in range(1, D // L):
                    acc += urows_v[i, pl.ds(k * L, L)] * irows_v[i, pl.ds(k * L, L)]
                res = jnp.where(lane == l, lane_sum(acc), res)
            out_v[pl.ds(base + gbase, L)] = res

    for c in range(NCH):
        fire(c)
        drain_and_compute(c)

    pltpu.sync_copy(out_v, out_hbm.at[wid])


@functools.partial(
    pl.kernel,
    out_type=jax.ShapeDtypeStruct((NW, BPW), jnp.float32),
    mesh=plsc.VectorSubcoreMesh(core_axis_name="c", subcore_axis_name="s"),
    scratch_types=[
        pltpu.VMEM((BPW,), jnp.int32),
        pltpu.VMEM((BPW,), jnp.int32),
        pltpu.VMEM((CH, D), jnp.float32),
        pltpu.VMEM((CH, D), jnp.float32),
        pltpu.VMEM((BPW,), jnp.float32),
        pltpu.SemaphoreType.DMA((NSEM,)),
    ],
)
def _two_tower(*args):
    _body(*args)


GT = 128


def _tc_body(uidx_s, iidx_s, utab, itab, out_ref, ubuf, ibuf, usem, isem):
    step = pl.program_id(0)
    base = step * GT

    @pl.loop(0, GT)
    def _fire(i):
        pltpu.make_async_copy(utab.at[uidx_s[base + i]], ubuf.at[i], usem).start()
        pltpu.make_async_copy(itab.at[iidx_s[base + i]], ibuf.at[i], isem).start()

    @pl.loop(0, GT)
    def _drain(i):
        pltpu.make_async_copy(utab.at[0], ubuf.at[0], usem).wait()
        pltpu.make_async_copy(itab.at[0], ibuf.at[0], isem).wait()

    out_ref[...] = jnp.sum(ubuf[...] * ibuf[...], axis=1)


_tc_tower = pl.pallas_call(
    _tc_body,
    grid_spec=pltpu.PrefetchScalarGridSpec(
        num_scalar_prefetch=2,
        grid=(B // GT,),
        in_specs=[
            pl.BlockSpec(memory_space=pl.ANY),
            pl.BlockSpec(memory_space=pl.ANY),
        ],
        out_specs=pl.BlockSpec((GT,), lambda i, *_: (i,)),
        scratch_shapes=[
            pltpu.VMEM((GT, D), jnp.float32),
            pltpu.VMEM((GT, D), jnp.float32),
            pltpu.SemaphoreType.DMA,
            pltpu.SemaphoreType.DMA,
        ],
    ),
    out_shape=jax.ShapeDtypeStruct((B,), jnp.float32),
)


def kernel(user_idx, item_idx, user_table, item_table):
    uidx = jnp.asarray(user_idx, jnp.int32)
    iidx = jnp.asarray(item_idx, jnp.int32)
    return _tc_tower(uidx, iidx, user_table, item_table)


# speedup vs baseline: 1.1123x; 1.1123x over previous
import functools

import jax
import jax.numpy as jnp
from jax import lax
from jax.experimental import pallas as pl
from jax.experimental.pallas import tpu as pltpu
from jax.experimental.pallas import tpu_sc as plsc

B = 16384
D = 64
NC = 2
NS = 16
NW = NC * NS
BPW = B // NW
CH = 128
NCH = BPW // CH
L = 16
NSEM = 8


def _body(uidx_hbm, iidx_hbm, utab_hbm, itab_hbm, out_hbm,
          uidx_v, iidx_v, urows_v, irows_v, out_v, sems):
    wid = lax.axis_index("s") * NC + lax.axis_index("c")

    pltpu.sync_copy(uidx_hbm.at[wid], uidx_v)
    pltpu.sync_copy(iidx_hbm.at[wid], iidx_v)

    lane = lax.iota(jnp.int32, L)

    def lane_sum(x):
        for k in (8, 4, 2, 1):
            x = x + x.at[lane ^ k].get(mode="promise_in_bounds", unique_indices=True)
        return x

    def fire(c):
        base = c * CH

        @pl.loop(0, CH // L)
        def _fire(g):
            uvec = uidx_v[pl.ds(base + g * L, L)]
            ivec = iidx_v[pl.ds(base + g * L, L)]
            for l in range(L):
                dst = g * L + l
                pltpu.async_copy(utab_hbm.at[uvec[l]], urows_v.at[dst],
                                 sems.at[(2 * l) % NSEM])
                pltpu.async_copy(itab_hbm.at[ivec[l]], irows_v.at[dst],
                                 sems.at[(2 * l + 1) % NSEM])

    def drain_and_compute(c):
        base = c * CH

        @pl.loop(0, CH // L)
        def _drain(g):
            for l in range(L):
                pltpu.make_async_copy(utab_hbm.at[0], urows_v.at[0],
                                      sems.at[(2 * l) % NSEM]).wait()
                pltpu.make_async_copy(itab_hbm.at[0], irows_v.at[0],
                                      sems.at[(2 * l + 1) % NSEM]).wait()

        @pl.loop(0, CH // L)
        def _group(g):
            gbase = g * L
            res = jnp.zeros((L,), jnp.float32)
            for l in range(L):
                i = gbase + l
                acc = urows_v[i, pl.ds(0, L)] * irows_v[i, pl.ds(0, L)]
                for k in range(1, D // L):
                    acc += urows_v[i, pl.ds(k * L, L)] * irows_v[i, pl.ds(k * L, L)]
                res = jnp.where(lane == l, lane_sum(acc), res)
            out_v[pl.ds(base + gbase, L)] = res

    for c in range(NCH):
        fire(c)
        drain_and_compute(c)

    pltpu.sync_copy(out_v, out_hbm.at[wid])


@functools.partial(
    pl.kernel,
    out_type=jax.ShapeDtypeStruct((NW, BPW), jnp.float32),
    mesh=plsc.VectorSubcoreMesh(core_axis_name="c", subcore_axis_name="s"),
    scratch_types=[
        pltpu.VMEM((BPW,), jnp.int32),
        pltpu.VMEM((BPW,), jnp.int32),
        pltpu.VMEM((CH, D), jnp.float32),
        pltpu.VMEM((CH, D), jnp.float32),
        pltpu.VMEM((BPW,), jnp.float32),
        pltpu.SemaphoreType.DMA((NSEM,)),
    ],
)
def _two_tower(*args):
    _body(*args)


GT = 512


def _tc_body(uidx_s, iidx_s, utab, itab, out_ref, ubuf, ibuf, usem, isem):
    step = pl.program_id(0)
    base = step * GT

    @pl.loop(0, GT, unroll=8)
    def _fire(i):
        pltpu.make_async_copy(utab.at[uidx_s[base + i]], ubuf.at[i], usem).start()
        pltpu.make_async_copy(itab.at[iidx_s[base + i]], ibuf.at[i], isem).start()

    @pl.loop(0, GT, unroll=8)
    def _drain(i):
        pltpu.make_async_copy(utab.at[0], ubuf.at[0], usem).wait()
        pltpu.make_async_copy(itab.at[0], ibuf.at[0], isem).wait()

    out_ref[...] = jnp.sum(ubuf[...] * ibuf[...], axis=1)


_tc_tower = pl.pallas_call(
    _tc_body,
    grid_spec=pltpu.PrefetchScalarGridSpec(
        num_scalar_prefetch=2,
        grid=(B // GT,),
        in_specs=[
            pl.BlockSpec(memory_space=pl.ANY),
            pl.BlockSpec(memory_space=pl.ANY),
        ],
        out_specs=pl.BlockSpec((GT,), lambda i, *_: (i,)),
        scratch_shapes=[
            pltpu.VMEM((GT, D), jnp.float32),
            pltpu.VMEM((GT, D), jnp.float32),
            pltpu.SemaphoreType.DMA,
            pltpu.SemaphoreType.DMA,
        ],
    ),
    out_shape=jax.ShapeDtypeStruct((B,), jnp.float32),
)


def kernel(user_idx, item_idx, user_table, item_table):
    uidx = jnp.asarray(user_idx, jnp.int32)
    iidx = jnp.asarray(item_idx, jnp.int32)
    return _tc_tower(uidx, iidx, user_table, item_table)


# speedup vs baseline: 1.1124x; 1.0000x over previous
import functools

import jax
import jax.numpy as jnp
from jax import lax
from jax.experimental import pallas as pl
from jax.experimental.pallas import tpu as pltpu
from jax.experimental.pallas import tpu_sc as plsc

B = 16384
D = 64
NC = 2
NS = 16
NW = NC * NS
BPW = B // NW
CH = 128
NCH = BPW // CH
L = 16
NSEM = 8


def _body(uidx_hbm, iidx_hbm, utab_hbm, itab_hbm, out_hbm,
          uidx_v, iidx_v, urows_v, irows_v, out_v, sems):
    wid = lax.axis_index("s") * NC + lax.axis_index("c")

    pltpu.sync_copy(uidx_hbm.at[wid], uidx_v)
    pltpu.sync_copy(iidx_hbm.at[wid], iidx_v)

    lane = lax.iota(jnp.int32, L)

    def lane_sum(x):
        for k in (8, 4, 2, 1):
            x = x + x.at[lane ^ k].get(mode="promise_in_bounds", unique_indices=True)
        return x

    def fire(c):
        base = c * CH

        @pl.loop(0, CH // L)
        def _fire(g):
            uvec = uidx_v[pl.ds(base + g * L, L)]
            ivec = iidx_v[pl.ds(base + g * L, L)]
            for l in range(L):
                dst = g * L + l
                pltpu.async_copy(utab_hbm.at[uvec[l]], urows_v.at[dst],
                                 sems.at[(2 * l) % NSEM])
                pltpu.async_copy(itab_hbm.at[ivec[l]], irows_v.at[dst],
                                 sems.at[(2 * l + 1) % NSEM])

    def drain_and_compute(c):
        base = c * CH

        @pl.loop(0, CH // L)
        def _drain(g):
            for l in range(L):
                pltpu.make_async_copy(utab_hbm.at[0], urows_v.at[0],
                                      sems.at[(2 * l) % NSEM]).wait()
                pltpu.make_async_copy(itab_hbm.at[0], irows_v.at[0],
                                      sems.at[(2 * l + 1) % NSEM]).wait()

        @pl.loop(0, CH // L)
        def _group(g):
            gbase = g * L
            res = jnp.zeros((L,), jnp.float32)
            for l in range(L):
                i = gbase + l
                acc = urows_v[i, pl.ds(0, L)] * irows_v[i, pl.ds(0, L)]
                for k in range(1, D // L):
                    acc += urows_v[i, pl.ds(k * L, L)] * irows_v[i, pl.ds(k * L, L)]
                res = jnp.where(lane == l, lane_sum(acc), res)
            out_v[pl.ds(base + gbase, L)] = res

    for c in range(NCH):
        fire(c)
        drain_and_compute(c)

    pltpu.sync_copy(out_v, out_hbm.at[wid])


@functools.partial(
    pl.kernel,
    out_type=jax.ShapeDtypeStruct((NW, BPW), jnp.float32),
    mesh=plsc.VectorSubcoreMesh(core_axis_name="c", subcore_axis_name="s"),
    scratch_types=[
        pltpu.VMEM((BPW,), jnp.int32),
        pltpu.VMEM((BPW,), jnp.int32),
        pltpu.VMEM((CH, D), jnp.float32),
        pltpu.VMEM((CH, D), jnp.float32),
        pltpu.VMEM((BPW,), jnp.float32),
        pltpu.SemaphoreType.DMA((NSEM,)),
    ],
)
def _two_tower(*args):
    _body(*args)


GT = 512


def _tc_body(uidx_s, iidx_s, utab, itab, out_ref, ubuf, ibuf, usem, isem):
    step = pl.program_id(0)
    base = step * GT

    @pl.loop(0, GT, unroll=8)
    def _fire(i):
        pltpu.make_async_copy(utab.at[uidx_s[base + i]], ubuf.at[i], usem).start()
        pltpu.make_async_copy(itab.at[iidx_s[base + i]], ibuf.at[i], isem).start()

    pltpu.make_async_copy(utab.at[pl.ds(0, GT), :], ubuf, usem).wait()
    pltpu.make_async_copy(itab.at[pl.ds(0, GT), :], ibuf, isem).wait()

    out_ref[...] = jnp.sum(ubuf[...] * ibuf[...], axis=1)


_tc_tower = pl.pallas_call(
    _tc_body,
    grid_spec=pltpu.PrefetchScalarGridSpec(
        num_scalar_prefetch=2,
        grid=(B // GT,),
        in_specs=[
            pl.BlockSpec(memory_space=pl.ANY),
            pl.BlockSpec(memory_space=pl.ANY),
        ],
        out_specs=pl.BlockSpec((GT,), lambda i, *_: (i,)),
        scratch_shapes=[
            pltpu.VMEM((GT, D), jnp.float32),
            pltpu.VMEM((GT, D), jnp.float32),
            pltpu.SemaphoreType.DMA,
            pltpu.SemaphoreType.DMA,
        ],
    ),
    out_shape=jax.ShapeDtypeStruct((B,), jnp.float32),
)


def kernel(user_idx, item_idx, user_table, item_table):
    uidx = jnp.asarray(user_idx, jnp.int32)
    iidx = jnp.asarray(item_idx, jnp.int32)
    return _tc_tower(uidx, iidx, user_table, item_table)


# speedup vs baseline: 1.2004x; 1.0791x over previous
import functools

import jax
import jax.numpy as jnp
from jax import lax
from jax.experimental import pallas as pl
from jax.experimental.pallas import tpu as pltpu
from jax.experimental.pallas import tpu_sc as plsc

B = 16384
D = 64
NC = 2
NS = 16
NW = NC * NS
L = 16
NSEM = 8
SB = 8192
GT = 512
TB = B - SB
BPW = SB // NW
CH = 128
NCH = BPW // CH


def _sc_body(uidx_hbm, iidx_hbm, utab_hbm, itab_hbm, out_hbm,
             uidx_v, iidx_v, urows_v, irows_v, out_v, sems):
    wid = lax.axis_index("s") * NC + lax.axis_index("c")

    pltpu.sync_copy(uidx_hbm.at[wid], uidx_v)
    pltpu.sync_copy(iidx_hbm.at[wid], iidx_v)

    lane = lax.iota(jnp.int32, L)

    def lane_sum(x):
        for k in (8, 4, 2, 1):
            x = x + x.at[lane ^ k].get(mode="promise_in_bounds", unique_indices=True)
        return x

    def fire(c):
        base = c * CH

        @pl.loop(0, CH // L)
        def _fire(g):
            uvec = uidx_v[pl.ds(base + g * L, L)]
            ivec = iidx_v[pl.ds(base + g * L, L)]
            for l in range(L):
                dst = g * L + l
                pltpu.async_copy(utab_hbm.at[uvec[l]], urows_v.at[dst],
                                 sems.at[(2 * l) % NSEM])
                pltpu.async_copy(itab_hbm.at[ivec[l]], irows_v.at[dst],
                                 sems.at[(2 * l + 1) % NSEM])

    def drain_and_compute(c):
        base = c * CH

        @pl.loop(0, CH // L)
        def _drain(g):
            for l in range(L):
                pltpu.make_async_copy(utab_hbm.at[0], urows_v.at[0],
                                      sems.at[(2 * l) % NSEM]).wait()
                pltpu.make_async_copy(itab_hbm.at[0], irows_v.at[0],
                                      sems.at[(2 * l + 1) % NSEM]).wait()

        @pl.loop(0, CH // L)
        def _group(g):
            gbase = g * L
            res = jnp.zeros((L,), jnp.float32)
            for l in range(L):
                i = gbase + l
                acc = urows_v[i, pl.ds(0, L)] * irows_v[i, pl.ds(0, L)]
                for k in range(1, D // L):
                    acc += urows_v[i, pl.ds(k * L, L)] * irows_v[i, pl.ds(k * L, L)]
                res = jnp.where(lane == l, lane_sum(acc), res)
            out_v[pl.ds(base + gbase, L)] = res

    for c in range(NCH):
        fire(c)
        drain_and_compute(c)

    pltpu.sync_copy(out_v, out_hbm.at[wid])


_sc_tower = functools.partial(
    pl.kernel,
    out_type=jax.ShapeDtypeStruct((NW, BPW), jnp.float32),
    mesh=plsc.VectorSubcoreMesh(core_axis_name="c", subcore_axis_name="s"),
    scratch_types=[
        pltpu.VMEM((BPW,), jnp.int32),
        pltpu.VMEM((BPW,), jnp.int32),
        pltpu.VMEM((CH, D), jnp.float32),
        pltpu.VMEM((CH, D), jnp.float32),
        pltpu.VMEM((BPW,), jnp.float32),
        pltpu.SemaphoreType.DMA((NSEM,)),
    ],
)(_sc_body)


def _tc_body(uidx_s, iidx_s, utab, itab, out_ref, ubuf, ibuf, usem, isem):
    step = pl.program_id(0)
    base = step * GT

    @pl.loop(0, GT, unroll=8)
    def _fire(i):
        pltpu.make_async_copy(utab.at[uidx_s[base + i]], ubuf.at[i], usem).start()
        pltpu.make_async_copy(itab.at[iidx_s[base + i]], ibuf.at[i], isem).start()

    pltpu.make_async_copy(utab.at[pl.ds(0, GT), :], ubuf, usem).wait()
    pltpu.make_async_copy(itab.at[pl.ds(0, GT), :], ibuf, isem).wait()

    out_ref[...] = jnp.sum(ubuf[...] * ibuf[...], axis=1)


_tc_tower = pl.pallas_call(
    _tc_body,
    grid_spec=pltpu.PrefetchScalarGridSpec(
        num_scalar_prefetch=2,
        grid=(TB // GT,),
        in_specs=[
            pl.BlockSpec(memory_space=pl.ANY),
            pl.BlockSpec(memory_space=pl.ANY),
        ],
        out_specs=pl.BlockSpec((GT,), lambda i, *_: (i,)),
        scratch_shapes=[
            pltpu.VMEM((GT, D), jnp.float32),
            pltpu.VMEM((GT, D), jnp.float32),
            pltpu.SemaphoreType.DMA,
            pltpu.SemaphoreType.DMA,
        ],
    ),
    out_shape=jax.ShapeDtypeStruct((TB,), jnp.float32),
)


def kernel(user_idx, item_idx, user_table, item_table):
    uidx = jnp.asarray(user_idx, jnp.int32)
    iidx = jnp.asarray(item_idx, jnp.int32)
    sc_out = _sc_tower(uidx[:SB].reshape(NW, BPW), iidx[:SB].reshape(NW, BPW),
                       user_table, item_table)
    tc_out = _tc_tower(uidx[SB:], iidx[SB:], user_table, item_table)
    return jnp.concatenate([sc_out.reshape(SB), tc_out])
